# Initial kernel scaffold; baseline (speedup 1.0000x reference)
import jax
import jax.numpy as jnp
from jax.experimental import pallas as pl


def kernel(x, graph_pool, padded_nei, adj, ope_ids, mask_job, mask_ope_mch, dur, a_index, old_action, mch_pooled, agv_pooled, params):
    raise NotImplementedError("write your pallas kernel here")


# speedup vs baseline: 6.4095x; 6.4095x over previous
import jax
import jax.numpy as jnp
from jax.experimental import pallas as pl
from jax.experimental.pallas import tpu as pltpu


def _matmul_body(a_ref, b_ref, o_ref):
    o_ref[...] = jnp.dot(a_ref[...], b_ref[...],
                         preferred_element_type=jnp.float32)


def _spmm(adj, h, br=256):
    tn = adj.shape[0]
    d = h.shape[1]
    return pl.pallas_call(
        _matmul_body,
        grid=(tn // br,),
        in_specs=[
            pl.BlockSpec((br, tn), lambda i: (i, 0)),
            pl.BlockSpec((tn, d), lambda i: (0, 0)),
        ],
        out_specs=pl.BlockSpec((br, d), lambda i: (i, 0)),
        out_shape=jax.ShapeDtypeStruct((tn, d), jnp.float32),
        compiler_params=pltpu.CompilerParams(
            dimension_semantics=("arbitrary",)),
    )(adj, h)


def _gin_body(p_ref, w1_ref, b1_ref, g1_ref, be1_ref,
              w2_ref, b2_ref, g2_ref, be2_ref, o_ref):
    z = jnp.dot(p_ref[...], w1_ref[...],
                preferred_element_type=jnp.float32) + b1_ref[...]
    mu = jnp.mean(z, axis=0, keepdims=True)
    var = jnp.mean((z - mu) ** 2, axis=0, keepdims=True)
    z = jnp.maximum(
        g1_ref[...] * (z - mu) / jnp.sqrt(var + 1e-5) + be1_ref[...], 0.0)
    z = jnp.dot(z, w2_ref[...],
                preferred_element_type=jnp.float32) + b2_ref[...]
    mu = jnp.mean(z, axis=0, keepdims=True)
    var = jnp.mean((z - mu) ** 2, axis=0, keepdims=True)
    o_ref[...] = jnp.maximum(
        g2_ref[...] * (z - mu) / jnp.sqrt(var + 1e-5) + be2_ref[...], 0.0)


def _gin_mlp(pooled, p, L):
    tn = pooled.shape[0]
    h = p[L + '_W1'].shape[1]
    row = lambda a: a.reshape(1, -1)
    return pl.pallas_call(
        _gin_body,
        out_shape=jax.ShapeDtypeStruct((tn, h), jnp.float32),
    )(pooled, p[L + '_W1'], row(p[L + '_b1']), row(p[L + '_bn1g']),
      row(p[L + '_bn1b']), p[L + '_W2'], row(p[L + '_b2']),
      row(p[L + '_bng']), row(p[L + '_bnb']))


def _actor_body(h_ref, gid_ref, ids_ref, mask_ref, gum_ref, mch_ref, agv_ref,
                dur_ref, w1_ref, b1_ref, w2_ref, b2_ref, w3_ref, b3_ref,
                aope_ref, job_ref, loga_ref, pt_ref, af_ref, hp_ref,
                fea_ref):
    tn, h = h_ref.shape
    b, nj = ids_ref.shape
    n = tn // b
    nm = dur_ref.shape[2]

    h3 = h_ref[...].reshape(b, n, h)
    hp = jnp.mean(h3, axis=1)
    hp_ref[...] = hp

    hh = h_ref[...]
    rows = b * nj
    chunk = 128
    for c in range(0, rows, chunk):
        m = min(chunk, rows - c)
        oh = (jax.lax.broadcasted_iota(jnp.int32, (m, tn), 1)
              == gid_ref[c:c + m, :]).astype(jnp.float32)
        fea_ref[c:c + m, :] = jnp.dot(oh, hh,
                                      preferred_element_type=jnp.float32)

    rep = lambda v: jnp.broadcast_to(v[:, None, :], (b, nj, h))
    feat = jnp.concatenate(
        [fea_ref[...].reshape(b, nj, h), rep(hp), rep(mch_ref[...]),
         rep(agv_ref[...])], axis=2).reshape(b * nj, 4 * h)
    z = jnp.tanh(jnp.dot(feat, w1_ref[...],
                         preferred_element_type=jnp.float32) + b1_ref[...])
    z = jnp.tanh(jnp.dot(z, w2_ref[...],
                         preferred_element_type=jnp.float32) + b2_ref[...])
    z3 = z.reshape(b, nj, h)
    scores = jnp.sum(z3 * w3_ref[...].reshape(1, 1, h), axis=2) * 10.0 \
        + b3_ref[0, 0] * 10.0
    logits = jnp.where(mask_ref[...] > 0.0, -jnp.inf, scores)

    y = logits + gum_ref[...]
    iota_j = jax.lax.broadcasted_iota(jnp.int32, (b, nj), 1)
    ymax = jnp.max(y, axis=1, keepdims=True)
    job = jnp.min(jnp.where(y == ymax, iota_j, nj), axis=1, keepdims=True)
    job_ref[...] = job

    lmax = jnp.max(logits, axis=1, keepdims=True)
    lse = lmax + jnp.log(jnp.sum(jnp.exp(logits - lmax), axis=1,
                                 keepdims=True))
    lsm = logits - lse
    oh_j = iota_j == job
    loga_ref[...] = jnp.sum(jnp.where(oh_j, lsm, 0.0), axis=1, keepdims=True)
    a_ope = jnp.sum(jnp.where(oh_j, ids_ref[...], 0), axis=1, keepdims=True)
    aope_ref[...] = a_ope

    oh_n = (jax.lax.broadcasted_iota(jnp.int32, (b, n), 1)
            == a_ope).astype(jnp.float32)
    pt_ref[...] = jnp.sum(dur_ref[...] * oh_n[:, :, None], axis=1)
    af_ref[...] = jnp.sum(h3 * oh_n[:, :, None], axis=1)


def _actor(h_nodes, ope_ids, mask_job, gum, mch_pooled, agv_pooled, dur, p):
    tn, h = h_nodes.shape
    b, nj = ope_ids.shape
    n = tn // b
    nm = dur.shape[2]
    row = lambda a: a.reshape(1, -1)
    gid = (ope_ids + n * jnp.arange(b, dtype=jnp.int32)[:, None]
           ).reshape(b * nj, 1)
    return pl.pallas_call(
        _actor_body,
        out_shape=(
            jax.ShapeDtypeStruct((b, 1), jnp.int32),
            jax.ShapeDtypeStruct((b, 1), jnp.int32),
            jax.ShapeDtypeStruct((b, 1), jnp.float32),
            jax.ShapeDtypeStruct((b, nm), jnp.float32),
            jax.ShapeDtypeStruct((b, h), jnp.float32),
            jax.ShapeDtypeStruct((b, h), jnp.float32),
        ),
        scratch_shapes=[pltpu.VMEM((b * nj, h), jnp.float32)],
    )(h_nodes, gid, ope_ids, mask_job.astype(jnp.float32), gum,
      mch_pooled, agv_pooled, dur, p['a_W1'], row(p['a_b1']), p['a_W2'],
      row(p['a_b2']), row(p['a_W3'][:, 0]), p['a_b3'].reshape(1, 1))


def kernel(x, graph_pool, padded_nei, adj, ope_ids, mask_job, mask_ope_mch,
           dur, a_index, old_action, mch_pooled, agv_pooled, params):
    p = params
    b, nj = ope_ids.shape
    nm = dur.shape[2]

    gum = jax.random.gumbel(jax.random.key(42), (b, nj), jnp.float32)

    pooled0 = _spmm(adj, x)
    h1 = _gin_mlp(pooled0, p, 'g0')
    pooled1 = _spmm(adj, h1)
    h2 = _gin_mlp(pooled1, p, 'g1')

    aope2, job2, loga2, pt_ope, act_feat, h_pooled = _actor(
        h2, ope_ids.astype(jnp.int32), mask_job, gum, mch_pooled,
        agv_pooled, dur, p)

    a_ope = aope2[:, 0]
    job_ids = job2[:, 0]
    log_a = loga2[:, 0]
    mask_mch_action = jnp.take_along_axis(
        mask_ope_mch,
        jnp.broadcast_to(a_ope[:, None, None], (b, 1, nm)), axis=1)
    return (a_ope, job_ids, log_a, pt_ope, act_feat, mask_mch_action,
            h_pooled)


# speedup vs baseline: 7.1640x; 1.1177x over previous
import jax
import jax.numpy as jnp
from jax.experimental import pallas as pl
from jax.experimental.pallas import tpu as pltpu


def _spmm_pack_body(a_ref, b_ref, o_ref, bm_ref):
    br, tn = a_ref.shape
    i = pl.program_id(0)
    a = a_ref[...]
    o_ref[...] = jnp.dot(a, b_ref[...], preferred_element_type=jnp.float32)
    row_g = jax.lax.broadcasted_iota(jnp.int32, (br, tn), 0) + i * br
    col = jax.lax.broadcasted_iota(jnp.int32, (br, tn), 1)
    thr = jnp.where(row_g == col, 1.5, 0.5)
    bits = (a > thr).astype(jnp.int32).reshape(br // 32, 32, tn)
    sh = jax.lax.broadcasted_iota(jnp.int32, (1, 32, 1), 1)
    bm_ref[...] = jnp.sum(jnp.left_shift(bits, sh), axis=1)


def _spmm_pack(adj, h, br=256):
    tn = adj.shape[0]
    d = h.shape[1]
    return pl.pallas_call(
        _spmm_pack_body,
        grid=(tn // br,),
        in_specs=[
            pl.BlockSpec((br, tn), lambda i: (i, 0)),
            pl.BlockSpec((tn, d), lambda i: (0, 0)),
        ],
        out_specs=[
            pl.BlockSpec((br, d), lambda i: (i, 0)),
            pl.BlockSpec((br // 32, tn), lambda i: (i, 0)),
        ],
        out_shape=[
            jax.ShapeDtypeStruct((tn, d), jnp.float32),
            jax.ShapeDtypeStruct((tn // 32, tn), jnp.int32),
        ],
        compiler_params=pltpu.CompilerParams(
            dimension_semantics=("arbitrary",)),
    )(adj, h)


def _spmm_unpack_body(bm_ref, b_ref, hblk_ref, o_ref):
    nw, tn = bm_ref.shape
    br = nw * 32
    sh = jax.lax.broadcasted_iota(jnp.int32, (1, 32, 1), 1)
    dense = jnp.bitwise_and(
        jnp.right_shift(bm_ref[...].reshape(nw, 1, tn), sh), 1
    ).astype(jnp.float32).reshape(br, tn)
    o_ref[...] = jnp.dot(dense, b_ref[...],
                         preferred_element_type=jnp.float32) + hblk_ref[...]


def _spmm_unpack(bitmap, h, br=256):
    tn = h.shape[0]
    d = h.shape[1]
    return pl.pallas_call(
        _spmm_unpack_body,
        grid=(tn // br,),
        in_specs=[
            pl.BlockSpec((br // 32, tn), lambda i: (i, 0)),
            pl.BlockSpec((tn, d), lambda i: (0, 0)),
            pl.BlockSpec((br, d), lambda i: (i, 0)),
        ],
        out_specs=pl.BlockSpec((br, d), lambda i: (i, 0)),
        out_shape=jax.ShapeDtypeStruct((tn, d), jnp.float32),
        compiler_params=pltpu.CompilerParams(
            dimension_semantics=("arbitrary",)),
    )(bitmap, h, h)


def _gin_body(p_ref, w1_ref, b1_ref, g1_ref, be1_ref,
              w2_ref, b2_ref, g2_ref, be2_ref, o_ref):
    z = jnp.dot(p_ref[...], w1_ref[...],
                preferred_element_type=jnp.float32) + b1_ref[...]
    mu = jnp.mean(z, axis=0, keepdims=True)
    var = jnp.mean((z - mu) ** 2, axis=0, keepdims=True)
    z = jnp.maximum(
        g1_ref[...] * (z - mu) / jnp.sqrt(var + 1e-5) + be1_ref[...], 0.0)
    z = jnp.dot(z, w2_ref[...],
                preferred_element_type=jnp.float32) + b2_ref[...]
    mu = jnp.mean(z, axis=0, keepdims=True)
    var = jnp.mean((z - mu) ** 2, axis=0, keepdims=True)
    o_ref[...] = jnp.maximum(
        g2_ref[...] * (z - mu) / jnp.sqrt(var + 1e-5) + be2_ref[...], 0.0)


def _gin_mlp(pooled, p, L):
    tn = pooled.shape[0]
    h = p[L + '_W1'].shape[1]
    row = lambda a: a.reshape(1, -1)
    return pl.pallas_call(
        _gin_body,
        out_shape=jax.ShapeDtypeStruct((tn, h), jnp.float32),
    )(pooled, p[L + '_W1'], row(p[L + '_b1']), row(p[L + '_bn1g']),
      row(p[L + '_bn1b']), p[L + '_W2'], row(p[L + '_b2']),
      row(p[L + '_bng']), row(p[L + '_bnb']))


def _actor_body(h_ref, gid_ref, ids_ref, mask_ref, gum_ref, mch_ref, agv_ref,
                dur_ref, w1_ref, b1_ref, w2_ref, b2_ref, w3_ref, b3_ref,
                aope_ref, job_ref, loga_ref, pt_ref, af_ref, hp_ref,
                fea_ref):
    tn, h = h_ref.shape
    b, nj = ids_ref.shape
    n = tn // b
    nm = dur_ref.shape[2]

    h3 = h_ref[...].reshape(b, n, h)
    hp = jnp.mean(h3, axis=1)
    hp_ref[...] = hp

    hh = h_ref[...]
    rows = b * nj
    chunk = 128
    for c in range(0, rows, chunk):
        m = min(chunk, rows - c)
        oh = (jax.lax.broadcasted_iota(jnp.int32, (m, tn), 1)
              == gid_ref[c:c + m, :]).astype(jnp.float32)
        fea_ref[c:c + m, :] = jnp.dot(oh, hh,
                                      preferred_element_type=jnp.float32)

    rep = lambda v: jnp.broadcast_to(v[:, None, :], (b, nj, h))
    feat = jnp.concatenate(
        [fea_ref[...].reshape(b, nj, h), rep(hp), rep(mch_ref[...]),
         rep(agv_ref[...])], axis=2).reshape(b * nj, 4 * h)
    z = jnp.tanh(jnp.dot(feat, w1_ref[...],
                         preferred_element_type=jnp.float32) + b1_ref[...])
    z = jnp.tanh(jnp.dot(z, w2_ref[...],
                         preferred_element_type=jnp.float32) + b2_ref[...])
    z3 = z.reshape(b, nj, h)
    scores = jnp.sum(z3 * w3_ref[...].reshape(1, 1, h), axis=2) * 10.0 \
        + b3_ref[0, 0] * 10.0
    logits = jnp.where(mask_ref[...] > 0.0, -jnp.inf, scores)

    y = logits + gum_ref[...]
    iota_j = jax.lax.broadcasted_iota(jnp.int32, (b, nj), 1)
    ymax = jnp.max(y, axis=1, keepdims=True)
    job = jnp.min(jnp.where(y == ymax, iota_j, nj), axis=1, keepdims=True)
    job_ref[...] = job

    lmax = jnp.max(logits, axis=1, keepdims=True)
    lse = lmax + jnp.log(jnp.sum(jnp.exp(logits - lmax), axis=1,
                                 keepdims=True))
    lsm = logits - lse
    oh_j = iota_j == job
    loga_ref[...] = jnp.sum(jnp.where(oh_j, lsm, 0.0), axis=1, keepdims=True)
    a_ope = jnp.sum(jnp.where(oh_j, ids_ref[...], 0), axis=1, keepdims=True)
    aope_ref[...] = a_ope

    oh_n = (jax.lax.broadcasted_iota(jnp.int32, (b, n), 1)
            == a_ope).astype(jnp.float32)
    pt_ref[...] = jnp.sum(dur_ref[...] * oh_n[:, :, None], axis=1)
    af_ref[...] = jnp.sum(h3 * oh_n[:, :, None], axis=1)


def _actor(h_nodes, ope_ids, mask_job, gum, mch_pooled, agv_pooled, dur, p):
    tn, h = h_nodes.shape
    b, nj = ope_ids.shape
    n = tn // b
    nm = dur.shape[2]
    row = lambda a: a.reshape(1, -1)
    gid = (ope_ids + n * jnp.arange(b, dtype=jnp.int32)[:, None]
           ).reshape(b * nj, 1)
    return pl.pallas_call(
        _actor_body,
        out_shape=(
            jax.ShapeDtypeStruct((b, 1), jnp.int32),
            jax.ShapeDtypeStruct((b, 1), jnp.int32),
            jax.ShapeDtypeStruct((b, 1), jnp.float32),
            jax.ShapeDtypeStruct((b, nm), jnp.float32),
            jax.ShapeDtypeStruct((b, h), jnp.float32),
            jax.ShapeDtypeStruct((b, h), jnp.float32),
        ),
        scratch_shapes=[pltpu.VMEM((b * nj, h), jnp.float32)],
    )(h_nodes, gid, ope_ids, mask_job.astype(jnp.float32), gum,
      mch_pooled, agv_pooled, dur, p['a_W1'], row(p['a_b1']), p['a_W2'],
      row(p['a_b2']), row(p['a_W3'][:, 0]), p['a_b3'].reshape(1, 1))


def kernel(x, graph_pool, padded_nei, adj, ope_ids, mask_job, mask_ope_mch,
           dur, a_index, old_action, mch_pooled, agv_pooled, params):
    p = params
    b, nj = ope_ids.shape
    nm = dur.shape[2]

    gum = jax.random.gumbel(jax.random.key(42), (b, nj), jnp.float32)

    pooled0, bitmap = _spmm_pack(adj, x)
    h1 = _gin_mlp(pooled0, p, 'g0')
    pooled1 = _spmm_unpack(bitmap, h1)
    h2 = _gin_mlp(pooled1, p, 'g1')

    aope2, job2, loga2, pt_ope, act_feat, h_pooled = _actor(
        h2, ope_ids.astype(jnp.int32), mask_job, gum, mch_pooled,
        agv_pooled, dur, p)

    a_ope = aope2[:, 0]
    job_ids = job2[:, 0]
    log_a = loga2[:, 0]
    mask_mch_action = jnp.take_along_axis(
        mask_ope_mch,
        jnp.broadcast_to(a_ope[:, None, None], (b, 1, nm)), axis=1)
    return (a_ope, job_ids, log_a, pt_ope, act_feat, mask_mch_action,
            h_pooled)
